# baseline (speedup 1.0000x reference)
import functools

import jax
import jax.numpy as jnp
from jax import lax
from jax.experimental import pallas as pl
from jax.experimental.pallas import tpu as pltpu
from jax.experimental.pallas import tpu_sc as plsc

_D = 768
_T = 77
_B = 4096
_BT = _B * _T

_NC = 2
_NS = 16
_NW = _NC * _NS
_PER_W = _BT // _NW
_CHUNK = 16
_NCH = _PER_W // _CHUNK
_LANES = 16
_ND = _D // _LANES


def _sc_embed(tokens_flat, table, pos):
    mesh = plsc.VectorSubcoreMesh(core_axis_name="c", subcore_axis_name="s")

    @functools.partial(
        pl.kernel,
        out_type=jax.ShapeDtypeStruct((_BT, _D), jnp.float32),
        mesh=mesh,
        scratch_types=[
            pltpu.VMEM((_PER_W,), jnp.int32),
            pltpu.VMEM((_T, _D), jnp.float32),
            pltpu.VMEM((_CHUNK, _D), jnp.float32),
            pltpu.SemaphoreType.DMA,
        ],
    )
    def run(tok_hbm, tab_hbm, pos_hbm, out_hbm, idx_v, pos_v, buf, gsem):
        wid = lax.axis_index("s") * _NC + lax.axis_index("c")
        base = wid * _PER_W
        pltpu.sync_copy(tok_hbm.at[pl.ds(base, _PER_W)], idx_v)
        pltpu.sync_copy(pos_hbm, pos_v)

        def step(j, carry):
            pltpu.async_copy(
                tab_hbm.at[idx_v.at[pl.ds(j * _CHUNK, _CHUNK)]], buf, gsem
            ).wait()

            def row(r, c):
                t = lax.rem(j * _CHUNK + r, _T)
                for dblk in range(_ND):
                    sl = pl.ds(dblk * _LANES, _LANES)
                    buf[r, sl] = buf[r, sl] + pos_v[t, sl]
                return c

            lax.fori_loop(0, _CHUNK, row, 0)
            pltpu.sync_copy(buf, out_hbm.at[pl.ds(base + j * _CHUNK, _CHUNK)])
            return carry

        lax.fori_loop(0, _NCH, step, 0)

    return run(tokens_flat, table, pos)


def kernel(tokens, token_embedding, position_embedding):
    idx = tokens.reshape(-1).astype(jnp.int32)
    out = _sc_embed(idx, token_embedding, position_embedding)
    return out.reshape(_B, _T, _D)


# speedup vs baseline: 1.5618x; 1.5618x over previous
import functools

import jax
import jax.numpy as jnp
from jax import lax
from jax.experimental import pallas as pl
from jax.experimental.pallas import tpu as pltpu
from jax.experimental.pallas import tpu_sc as plsc

_D = 768
_T = 77
_B = 4096
_BT = _B * _T

_NC = 2
_NS = 16
_NW = _NC * _NS
_PER_W = _BT // _NW
_CHUNK = 16
_NCH = _PER_W // _CHUNK
_NBUF = 4
_LANES = 16
_ND = _D // _LANES


def _sc_embed(tokens_flat, table, pos):
    mesh = plsc.VectorSubcoreMesh(core_axis_name="c", subcore_axis_name="s")

    @functools.partial(
        pl.kernel,
        out_type=jax.ShapeDtypeStruct((_BT, _D), jnp.float32),
        mesh=mesh,
        scratch_types=[
            pltpu.VMEM((_PER_W,), jnp.int32),
            pltpu.VMEM((_T, _D), jnp.float32),
        ]
        + [pltpu.VMEM((_CHUNK, _D), jnp.float32)] * _NBUF
        + [pltpu.SemaphoreType.DMA] * (2 * _NBUF),
    )
    def run(tok_hbm, tab_hbm, pos_hbm, out_hbm, idx_v, pos_v, *rest):
        bufs = rest[:_NBUF]
        gsems = rest[_NBUF:2 * _NBUF]
        ssems = rest[2 * _NBUF:]

        wid = lax.axis_index("s") * _NC + lax.axis_index("c")
        base = wid * _PER_W
        pltpu.sync_copy(tok_hbm.at[pl.ds(base, _PER_W)], idx_v)
        pltpu.sync_copy(pos_hbm, pos_v)

        def start_gather(j, slot):
            pltpu.async_copy(
                tab_hbm.at[idx_v.at[pl.ds(j * _CHUNK, _CHUNK)]],
                bufs[slot],
                gsems[slot],
            )

        start_gather(0, 0)
        start_gather(1, 1)

        def outer(i, carry):
            for b in range(_NBUF):
                j = i * _NBUF + b
                nslot = (b + 2) % _NBUF

                @pl.when(j + 2 < _NCH)
                def _():
                    @pl.when(j >= 2)
                    def _():
                        pltpu.make_async_copy(
                            bufs[nslot],
                            out_hbm.at[pl.ds(0, _CHUNK)],
                            ssems[nslot],
                        ).wait()

                    start_gather(j + 2, nslot)

                pltpu.make_async_copy(
                    tab_hbm.at[pl.ds(0, _CHUNK)],
                    bufs[b],
                    gsems[b],
                ).wait()

                def row(r, c):
                    t = lax.rem(j * _CHUNK + r, _T)
                    for dblk in range(_ND):
                        sl = pl.ds(dblk * _LANES, _LANES)
                        plsc.addupdate(bufs[b].at[r, sl], pos_v[t, sl])
                    return c

                lax.fori_loop(0, _CHUNK, row, 0)

                pltpu.async_copy(
                    bufs[b],
                    out_hbm.at[pl.ds(base + j * _CHUNK, _CHUNK)],
                    ssems[b],
                )
            return carry

        lax.fori_loop(0, _NCH // _NBUF, outer, 0)

        for b in range(_NBUF):
            pltpu.make_async_copy(
                bufs[b], out_hbm.at[pl.ds(0, _CHUNK)], ssems[b]
            ).wait()

    return run(tokens_flat, table, pos)


def kernel(tokens, token_embedding, position_embedding):
    idx = tokens.reshape(-1).astype(jnp.int32)
    out = _sc_embed(idx, token_embedding, position_embedding)
    return out.reshape(_B, _T, _D)


# speedup vs baseline: 2.0006x; 1.2810x over previous
import functools

import jax
import jax.numpy as jnp
from jax import lax
from jax.experimental import pallas as pl
from jax.experimental.pallas import tpu as pltpu
from jax.experimental.pallas import tpu_sc as plsc

_D = 768
_T = 77
_B = 4096
_BT = _B * _T

_NC = 2
_NS = 16
_NW = _NC * _NS
_PER_W = _BT // _NW
_CHUNK = 16
_NCH = _PER_W // _CHUNK
_NBUF = 4
_LANES = 16
_ND = _D // _LANES


def _sc_embed(tokens_flat, table, pos):
    mesh = plsc.VectorSubcoreMesh(core_axis_name="c", subcore_axis_name="s")

    @functools.partial(
        pl.kernel,
        out_type=jax.ShapeDtypeStruct((_BT, _D), jnp.float32),
        mesh=mesh,
        scratch_types=[
            pltpu.VMEM((_PER_W,), jnp.int32),
            pltpu.VMEM((_T, _D), jnp.float32),
        ]
        + [pltpu.VMEM((_CHUNK, _D), jnp.float32)] * _NBUF
        + [pltpu.SemaphoreType.DMA] * (2 * _NBUF),
    )
    def run(tok_hbm, tab_hbm, pos_hbm, out_hbm, idx_v, pos_v, *rest):
        bufs = rest[:_NBUF]
        gsems = rest[_NBUF:2 * _NBUF]
        ssems = rest[2 * _NBUF:]

        wid = lax.axis_index("s") * _NC + lax.axis_index("c")
        base = wid * _PER_W
        pltpu.sync_copy(tok_hbm.at[pl.ds(base, _PER_W)], idx_v)
        pltpu.sync_copy(pos_hbm, pos_v)

        def start_gather(j, slot):
            pltpu.async_copy(
                tab_hbm.at[idx_v.at[pl.ds(j * _CHUNK, _CHUNK)]],
                bufs[slot],
                gsems[slot],
            )

        start_gather(0, 0)
        start_gather(1, 1)

        def outer(i, carry):
            for b in range(_NBUF):
                j = i * _NBUF + b
                nslot = (b + 2) % _NBUF

                @pl.when(j + 2 < _NCH)
                def _():
                    @pl.when(j >= 2)
                    def _():
                        pltpu.make_async_copy(
                            bufs[nslot],
                            out_hbm.at[pl.ds(0, _CHUNK)],
                            ssems[nslot],
                        ).wait()

                    start_gather(j + 2, nslot)

                pltpu.make_async_copy(
                    tab_hbm.at[pl.ds(0, _CHUNK)],
                    bufs[b],
                    gsems[b],
                ).wait()

                @plsc.parallel_loop(0, _CHUNK, 1, unroll=2)
                def row(r):
                    t = lax.rem(j * _CHUNK + r, _T)
                    for dblk in range(_ND):
                        sl = pl.ds(dblk * _LANES, _LANES)
                        plsc.addupdate(bufs[b].at[r, sl], pos_v[t, sl])

                pltpu.async_copy(
                    bufs[b],
                    out_hbm.at[pl.ds(base + j * _CHUNK, _CHUNK)],
                    ssems[b],
                )
            return carry

        lax.fori_loop(0, _NCH // _NBUF, outer, 0)

        for b in range(_NBUF):
            pltpu.make_async_copy(
                bufs[b], out_hbm.at[pl.ds(0, _CHUNK)], ssems[b]
            ).wait()

    return run(tokens_flat, table, pos)


def kernel(tokens, token_embedding, position_embedding):
    idx = tokens.reshape(-1).astype(jnp.int32)
    out = _sc_embed(idx, token_embedding, position_embedding)
    return out.reshape(_B, _T, _D)


# speedup vs baseline: 2.3260x; 1.1626x over previous
import functools

import jax
import jax.numpy as jnp
from jax import lax
from jax.experimental import pallas as pl
from jax.experimental.pallas import tpu as pltpu
from jax.experimental.pallas import tpu_sc as plsc

_D = 768
_T = 77
_B = 4096
_BT = _B * _T

_NC = 2
_NS = 16
_NW = _NC * _NS
_SAMP_W = _B // _NW
_PER_W = _BT // _NW
_CHUNK = 16
_NCC = _SAMP_W // _CHUNK
_NCH = _T * _NCC
_NBUF = 4
_LANES = 16
_ND = _D // _LANES


def _sc_embed(tokens_flat, table, pos):
    mesh = plsc.VectorSubcoreMesh(core_axis_name="c", subcore_axis_name="s")

    @functools.partial(
        pl.kernel,
        out_type=jax.ShapeDtypeStruct((_BT, _D), jnp.float32),
        mesh=mesh,
        scratch_types=[
            pltpu.VMEM((_T, _SAMP_W), jnp.int32),
            pltpu.VMEM((_T, _D), jnp.float32),
        ]
        + [pltpu.VMEM((_CHUNK, _D), jnp.float32)] * _NBUF
        + [pltpu.SemaphoreType.DMA] * (2 * _NBUF),
    )
    def run(tok_hbm, tab_hbm, pos_hbm, out_hbm, idx_v, pos_v, *rest):
        bufs = rest[:_NBUF]
        gsems = rest[_NBUF:2 * _NBUF]
        ssems = rest[2 * _NBUF:]

        wid = lax.axis_index("s") * _NC + lax.axis_index("c")
        base = wid * _PER_W
        pltpu.sync_copy(tok_hbm.at[:, pl.ds(wid * _SAMP_W, _SAMP_W)], idx_v)
        pltpu.sync_copy(pos_hbm, pos_v)

        iota77 = lax.iota(jnp.int32, _LANES) * _T

        def chunk_off(j):
            t = j // _NCC
            c = lax.rem(j, _NCC)
            return t, c, c * (_CHUNK * _T) + t

        def start_gather(j, slot):
            t, c, _ = chunk_off(j)
            tokvec = idx_v[t, pl.ds(c * _CHUNK, _CHUNK)]
            pltpu.async_copy(tab_hbm.at[tokvec], bufs[slot], gsems[slot])

        start_gather(0, 0)
        start_gather(1, 1)

        def outer(i, carry):
            for b in range(_NBUF):
                j = i * _NBUF + b
                nslot = (b + 2) % _NBUF

                @pl.when(j + 2 < _NCH)
                def _():
                    @pl.when(j >= 2)
                    def _():
                        pltpu.make_async_copy(
                            bufs[nslot],
                            out_hbm.at[pl.ds(0, _CHUNK)],
                            ssems[nslot],
                        ).wait()

                    start_gather(j + 2, nslot)

                pltpu.make_async_copy(
                    tab_hbm.at[pl.ds(0, _CHUNK)],
                    bufs[b],
                    gsems[b],
                ).wait()

                t, _, off = chunk_off(j)
                buf = bufs[b]

                @plsc.parallel_loop(0, _ND, 1, unroll=2)
                def dcol(dblk):
                    sl = pl.ds(dblk * _LANES, _LANES)
                    p = pos_v[t, sl]
                    for r in range(_CHUNK):
                        plsc.addupdate(buf.at[r, sl], p)

                ovec = iota77 + (base + off)
                pltpu.async_copy(bufs[b], out_hbm.at[ovec], ssems[b])
            return carry

        lax.fori_loop(0, _NCH // _NBUF, outer, 0)

        for b in range(_NBUF):
            pltpu.make_async_copy(
                bufs[b], out_hbm.at[pl.ds(0, _CHUNK)], ssems[b]
            ).wait()

    return run(tokens_flat, table, pos)


def kernel(tokens, token_embedding, position_embedding):
    idx_t = tokens.astype(jnp.int32).T
    out = _sc_embed(idx_t, token_embedding, position_embedding)
    return out.reshape(_B, _T, _D)
